# baseline (speedup 1.0000x reference)
import jax
import jax.numpy as jnp
from jax import lax
from jax.experimental import pallas as pl
from jax.experimental.pallas import tpu as pltpu
from jax.experimental.pallas import tpu_sc as plsc

N_STYLE = 1000000
D_STYLE = 64
BATCH = 16384
HIST = 20

NC = 2
NS = 16
NW = NC * NS
LANES = 16

NIDX = BATCH * HIST
N_PER_W = NIDX // NW

CHUNK = 256
N_CHUNKS = N_PER_W // CHUNK

TCOLS = 512
TBLKS = (N_STYLE + TCOLS - 1) // TCOLS


def _relayout_body(lut_t_ref, table_ref):
    table_ref[:, :D_STYLE] = lut_t_ref[...].T


def _gather_body(idx_hbm, table_hbm, out_hbm, idx_v, rows_v, sem):
    wid = lax.axis_index("s") * NC + lax.axis_index("c")
    base = wid * N_PER_W
    pltpu.sync_copy(idx_hbm.at[pl.ds(base, N_PER_W)], idx_v)

    def chunk_step(k, carry):
        off = k * CHUNK
        idx_chunk = idx_v.at[pl.ds(off, CHUNK)]
        pltpu.async_copy(table_hbm.at[idx_chunk], rows_v, sem).wait()
        pltpu.sync_copy(rows_v, out_hbm.at[pl.ds(base + off, CHUNK)])
        return carry

    lax.fori_loop(0, N_CHUNKS, chunk_step, 0)


@jax.jit
def _embed(x_flat, lut_t):
    table = pl.pallas_call(
        _relayout_body,
        grid=(TBLKS,),
        in_specs=[pl.BlockSpec((D_STYLE, TCOLS), lambda i: (0, i))],
        out_specs=pl.BlockSpec((TCOLS, 2 * D_STYLE), lambda i: (i, 0)),
        out_shape=jax.ShapeDtypeStruct((N_STYLE, 2 * D_STYLE), jnp.float32),
    )(lut_t)

    mesh = plsc.VectorSubcoreMesh(
        core_axis_name="c", subcore_axis_name="s", num_cores=NC,
        num_subcores=NS)
    gather_k = pl.kernel(
        _gather_body,
        out_type=jax.ShapeDtypeStruct((NIDX, 2 * D_STYLE), jnp.float32),
        mesh=mesh,
        scratch_types=[
            pltpu.VMEM((N_PER_W,), jnp.int32),
            pltpu.VMEM((CHUNK, 2 * D_STYLE), jnp.float32),
            pltpu.SemaphoreType.DMA,
        ],
        compiler_params=pltpu.CompilerParams(needs_layout_passes=False),
    )
    out_pad = gather_k(x_flat, table)
    return out_pad[:, :D_STYLE]


def kernel(x, lut):
    x_flat = x.reshape(NIDX).astype(jnp.int32)
    out = _embed(x_flat, lut.T)
    return out.reshape(BATCH, HIST, D_STYLE)


# speedup vs baseline: 1.3847x; 1.3847x over previous
import jax
import jax.numpy as jnp
from jax import lax
from jax.experimental import pallas as pl
from jax.experimental.pallas import tpu as pltpu
from jax.experimental.pallas import tpu_sc as plsc

N_STYLE = 1000000
D_STYLE = 64
BATCH = 16384
HIST = 20

NC = 2
NS = 16
NW = NC * NS
LANES = 16

NIDX = BATCH * HIST
N_PER_W = NIDX // NW

CHUNK = 256
N_CHUNKS = N_PER_W // CHUNK

TCOLS = 1024
TBLKS = (N_STYLE + TCOLS - 1) // TCOLS


def _relayout_body(lut_t_ref, table_ref):
    row = lax.broadcasted_iota(jnp.int32, (D_STYLE, 2 * D_STYLE), 0)
    col = lax.broadcasted_iota(jnp.int32, (D_STYLE, 2 * D_STYLE), 1)
    eye = (row == col).astype(jnp.float32)
    table_ref[...] = lax.dot_general(
        lut_t_ref[...], eye, (((0,), (0,)), ((), ())),
        preferred_element_type=jnp.float32)


def _gather_body(idx_hbm, table_hbm, out_hbm, idx_v, rows_v, sem):
    wid = lax.axis_index("s") * NC + lax.axis_index("c")
    base = wid * N_PER_W
    pltpu.sync_copy(idx_hbm.at[pl.ds(base, N_PER_W)], idx_v)

    def chunk_step(k, carry):
        off = k * CHUNK
        idx_chunk = idx_v.at[pl.ds(off, CHUNK)]
        pltpu.async_copy(table_hbm.at[idx_chunk], rows_v, sem).wait()
        pltpu.sync_copy(rows_v, out_hbm.at[pl.ds(base + off, CHUNK)])
        return carry

    lax.fori_loop(0, N_CHUNKS, chunk_step, 0)


@jax.jit
def _embed(x_flat, lut_t):
    table = pl.pallas_call(
        _relayout_body,
        grid=(TBLKS,),
        in_specs=[pl.BlockSpec((D_STYLE, TCOLS), lambda i: (0, i))],
        out_specs=pl.BlockSpec((TCOLS, 2 * D_STYLE), lambda i: (i, 0)),
        out_shape=jax.ShapeDtypeStruct((N_STYLE, 2 * D_STYLE), jnp.float32),
    )(lut_t)

    mesh = plsc.VectorSubcoreMesh(
        core_axis_name="c", subcore_axis_name="s", num_cores=NC,
        num_subcores=NS)
    gather_k = pl.kernel(
        _gather_body,
        out_type=jax.ShapeDtypeStruct((NIDX, 2 * D_STYLE), jnp.float32),
        mesh=mesh,
        scratch_types=[
            pltpu.VMEM((N_PER_W,), jnp.int32),
            pltpu.VMEM((CHUNK, 2 * D_STYLE), jnp.float32),
            pltpu.SemaphoreType.DMA,
        ],
        compiler_params=pltpu.CompilerParams(needs_layout_passes=False),
    )
    out_pad = gather_k(x_flat, table)
    return out_pad[:, :D_STYLE]


def kernel(x, lut):
    x_flat = x.reshape(NIDX).astype(jnp.int32)
    out = _embed(x_flat, lut.T)
    return out.reshape(BATCH, HIST, D_STYLE)


# speedup vs baseline: 1.7989x; 1.2991x over previous
import jax
import jax.numpy as jnp
from jax import lax
from jax.experimental import pallas as pl
from jax.experimental.pallas import tpu as pltpu
from jax.experimental.pallas import tpu_sc as plsc

N_STYLE = 1000000
D_STYLE = 64
BATCH = 16384
HIST = 20

NC = 2
NS = 16
NW = NC * NS
LANES = 16

NIDX = BATCH * HIST
N_PER_W = NIDX // NW

CHUNK = 256
N_CHUNKS = N_PER_W // CHUNK

TCOLS = 4096
TBLKS = (N_STYLE + TCOLS - 1) // TCOLS


def _relayout_body(lut_t_ref, table_ref):
    row = lax.broadcasted_iota(jnp.int32, (D_STYLE, 2 * D_STYLE), 0)
    col = lax.broadcasted_iota(jnp.int32, (D_STYLE, 2 * D_STYLE), 1)
    eye = (row == col).astype(jnp.float32)
    table_ref[...] = lax.dot_general(
        lut_t_ref[...], eye, (((0,), (0,)), ((), ())),
        precision=lax.Precision.HIGHEST,
        preferred_element_type=jnp.float32)


def _gather_body(idx_hbm, table_hbm, out_hbm, idx_v, rows_v, sem):
    wid = lax.axis_index("s") * NC + lax.axis_index("c")
    base = wid * N_PER_W
    pltpu.sync_copy(idx_hbm.at[pl.ds(base, N_PER_W)], idx_v)

    def chunk_step(k, carry):
        off = k * CHUNK
        idx_chunk = idx_v.at[pl.ds(off, CHUNK)]
        pltpu.async_copy(table_hbm.at[idx_chunk], rows_v, sem).wait()
        pltpu.sync_copy(rows_v, out_hbm.at[pl.ds(base + off, CHUNK)])
        return carry

    lax.fori_loop(0, N_CHUNKS, chunk_step, 0)


@jax.jit
def _embed(x_flat, lut_t):
    table = pl.pallas_call(
        _relayout_body,
        grid=(TBLKS,),
        in_specs=[pl.BlockSpec((D_STYLE, TCOLS), lambda i: (0, i))],
        out_specs=pl.BlockSpec((TCOLS, 2 * D_STYLE), lambda i: (i, 0)),
        out_shape=jax.ShapeDtypeStruct((N_STYLE, 2 * D_STYLE), jnp.float32),
    )(lut_t)

    mesh = plsc.VectorSubcoreMesh(
        core_axis_name="c", subcore_axis_name="s", num_cores=NC,
        num_subcores=NS)
    gather_k = pl.kernel(
        _gather_body,
        out_type=jax.ShapeDtypeStruct((NIDX, 2 * D_STYLE), jnp.float32),
        mesh=mesh,
        scratch_types=[
            pltpu.VMEM((N_PER_W,), jnp.int32),
            pltpu.VMEM((CHUNK, 2 * D_STYLE), jnp.float32),
            pltpu.SemaphoreType.DMA,
        ],
        compiler_params=pltpu.CompilerParams(needs_layout_passes=False),
    )
    out_pad = gather_k(x_flat, table)
    return out_pad[:, :D_STYLE]


def kernel(x, lut):
    x_flat = x.reshape(NIDX).astype(jnp.int32)
    out = _embed(x_flat, lut.T)
    return out.reshape(BATCH, HIST, D_STYLE)


# speedup vs baseline: 1.7999x; 1.0005x over previous
import jax
import jax.numpy as jnp
from jax import lax
from jax.experimental import pallas as pl
from jax.experimental.pallas import tpu as pltpu
from jax.experimental.pallas import tpu_sc as plsc

N_STYLE = 1000000
D_STYLE = 64
BATCH = 16384
HIST = 20

NC = 2
NS = 16
NW = NC * NS
LANES = 16

NIDX = BATCH * HIST
N_PER_W = NIDX // NW

CHUNK = 256
N_CHUNKS = N_PER_W // CHUNK

TCOLS = 4096
TBLKS = (N_STYLE + TCOLS - 1) // TCOLS


def _relayout_body(lut_t_ref, table_ref):
    row = lax.broadcasted_iota(jnp.int32, (D_STYLE, 2 * D_STYLE), 0)
    col = lax.broadcasted_iota(jnp.int32, (D_STYLE, 2 * D_STYLE), 1)
    eye = (row == col).astype(jnp.float32)
    table_ref[...] = lax.dot_general(
        lut_t_ref[...], eye, (((0,), (0,)), ((), ())),
        precision=lax.Precision.HIGHEST,
        preferred_element_type=jnp.float32)


def _gather_body(idx_hbm, table_hbm, out_hbm, idx_v, rows_v, sem):
    wid = lax.axis_index("s") * NC + lax.axis_index("c")
    base = wid * N_PER_W
    pltpu.sync_copy(idx_hbm.at[pl.ds(base, N_PER_W)], idx_v)

    def chunk_step(k, carry):
        off = k * CHUNK
        idx_chunk = idx_v.at[pl.ds(off, CHUNK)]
        pltpu.async_copy(table_hbm.at[idx_chunk], rows_v, sem).wait()
        pltpu.sync_copy(rows_v, out_hbm.at[pl.ds(base + off, CHUNK)])
        return carry

    lax.fori_loop(0, N_CHUNKS, chunk_step, 0)


@jax.jit
def _embed(x_flat, lut_t):
    table = pl.pallas_call(
        _relayout_body,
        grid=(TBLKS,),
        in_specs=[pl.BlockSpec((D_STYLE, TCOLS), lambda i: (0, i))],
        out_specs=pl.BlockSpec((TCOLS, 2 * D_STYLE), lambda i: (i, 0)),
        out_shape=jax.ShapeDtypeStruct((N_STYLE, 2 * D_STYLE), jnp.float32),
    )(lut_t)

    mesh = plsc.VectorSubcoreMesh(
        core_axis_name="c", subcore_axis_name="s", num_cores=NC,
        num_subcores=NS)
    gather_k = pl.kernel(
        _gather_body,
        out_type=jax.ShapeDtypeStruct((NIDX, 2 * D_STYLE), jnp.float32),
        mesh=mesh,
        scratch_types=[
            pltpu.VMEM((N_PER_W,), jnp.int32),
            pltpu.VMEM((CHUNK, 2 * D_STYLE), jnp.float32),
            pltpu.SemaphoreType.DMA,
        ],
        compiler_params=pltpu.CompilerParams(needs_layout_passes=False),
    )
    out_pad = gather_k(x_flat, table)
    return out_pad.reshape(BATCH, HIST, 2 * D_STYLE)[:, :, :D_STYLE]


def kernel(x, lut):
    x_flat = x.reshape(NIDX).astype(jnp.int32)
    return _embed(x_flat, lut.T)


# speedup vs baseline: 2.0055x; 1.1142x over previous
import jax
import jax.numpy as jnp
from jax import lax
from jax.experimental import pallas as pl
from jax.experimental.pallas import tpu as pltpu
from jax.experimental.pallas import tpu_sc as plsc

N_STYLE = 1000000
D_STYLE = 64
BATCH = 16384
HIST = 20

NC = 2
NS = 16
NW = NC * NS
LANES = 16

NIDX = BATCH * HIST
N_PER_W = NIDX // NW

CHUNK = 256
N_CHUNKS = N_PER_W // CHUNK

TCOLS = 4096
TBLKS = (N_STYLE + TCOLS - 1) // TCOLS


def _relayout_body(lut_t_ref, table_ref):
    row = lax.broadcasted_iota(jnp.int32, (D_STYLE, 2 * D_STYLE), 0)
    col = lax.broadcasted_iota(jnp.int32, (D_STYLE, 2 * D_STYLE), 1)
    eye = (row == col).astype(jnp.float32)
    x = lut_t_ref[...]
    hi = x.astype(jnp.bfloat16).astype(jnp.float32)
    lo = x - hi
    dims = (((0,), (0,)), ((), ()))
    table_ref[...] = (
        lax.dot_general(hi, eye, dims, preferred_element_type=jnp.float32)
        + lax.dot_general(lo, eye, dims, preferred_element_type=jnp.float32))


def _gather_body(idx_hbm, table_hbm, out_hbm, idx_v, rows_v, sem):
    wid = lax.axis_index("s") * NC + lax.axis_index("c")
    base = wid * N_PER_W
    pltpu.sync_copy(idx_hbm.at[pl.ds(base, N_PER_W)], idx_v)

    def chunk_step(k, carry):
        off = k * CHUNK
        idx_chunk = idx_v.at[pl.ds(off, CHUNK)]
        pltpu.async_copy(table_hbm.at[idx_chunk], rows_v, sem).wait()
        pltpu.sync_copy(rows_v, out_hbm.at[pl.ds(base + off, CHUNK)])
        return carry

    lax.fori_loop(0, N_CHUNKS, chunk_step, 0)


@jax.jit
def _embed(x_flat, lut_t):
    table = pl.pallas_call(
        _relayout_body,
        grid=(TBLKS,),
        in_specs=[pl.BlockSpec((D_STYLE, TCOLS), lambda i: (0, i))],
        out_specs=pl.BlockSpec((TCOLS, 2 * D_STYLE), lambda i: (i, 0)),
        out_shape=jax.ShapeDtypeStruct((N_STYLE, 2 * D_STYLE), jnp.float32),
    )(lut_t)

    mesh = plsc.VectorSubcoreMesh(
        core_axis_name="c", subcore_axis_name="s", num_cores=NC,
        num_subcores=NS)
    gather_k = pl.kernel(
        _gather_body,
        out_type=jax.ShapeDtypeStruct((NIDX, 2 * D_STYLE), jnp.float32),
        mesh=mesh,
        scratch_types=[
            pltpu.VMEM((N_PER_W,), jnp.int32),
            pltpu.VMEM((CHUNK, 2 * D_STYLE), jnp.float32),
            pltpu.SemaphoreType.DMA,
        ],
        compiler_params=pltpu.CompilerParams(needs_layout_passes=False),
    )
    out_pad = gather_k(x_flat, table)
    return out_pad.reshape(BATCH, HIST, 2 * D_STYLE)[:, :, :D_STYLE]


def kernel(x, lut):
    x_flat = x.reshape(NIDX).astype(jnp.int32)
    return _embed(x_flat, lut.T)


# speedup vs baseline: 2.1959x; 1.0950x over previous
import jax
import jax.numpy as jnp
from jax import lax
from jax.experimental import pallas as pl
from jax.experimental.pallas import tpu as pltpu
from jax.experimental.pallas import tpu_sc as plsc

N_STYLE = 1000000
D_STYLE = 64
BATCH = 16384
HIST = 20

NC = 2
NS = 16
NW = NC * NS
LANES = 16

NIDX = BATCH * HIST
N_PER_W = NIDX // NW

CHUNK = 256
N_CHUNKS = N_PER_W // CHUNK

TCOLS = 8192
TBLKS = (N_STYLE + TCOLS - 1) // TCOLS


def _relayout_body(lut_t_ref, table_ref):
    row = lax.broadcasted_iota(jnp.int32, (D_STYLE, 2 * D_STYLE), 0)
    col = lax.broadcasted_iota(jnp.int32, (D_STYLE, 2 * D_STYLE), 1)
    eye = (row == col).astype(jnp.float32)
    x = lut_t_ref[...]
    hi = x.astype(jnp.bfloat16).astype(jnp.float32)
    lo = x - hi
    dims = (((0,), (0,)), ((), ()))
    table_ref[...] = (
        lax.dot_general(hi, eye, dims, preferred_element_type=jnp.float32)
        + lax.dot_general(lo, eye, dims, preferred_element_type=jnp.float32))


def _gather_body(idx_hbm, table_hbm, out_hbm, idx_v, rows_v, sem):
    wid = lax.axis_index("s") * NC + lax.axis_index("c")
    base = wid * N_PER_W
    pltpu.sync_copy(idx_hbm.at[pl.ds(base, N_PER_W)], idx_v)

    def chunk_step(k, carry):
        off = k * CHUNK
        idx_chunk = idx_v.at[pl.ds(off, CHUNK)]
        pltpu.async_copy(table_hbm.at[idx_chunk], rows_v, sem).wait()
        pltpu.sync_copy(rows_v, out_hbm.at[pl.ds(base + off, CHUNK)])
        return carry

    lax.fori_loop(0, N_CHUNKS, chunk_step, 0)


@jax.jit
def _embed(x_flat, lut_t):
    table = pl.pallas_call(
        _relayout_body,
        grid=(TBLKS,),
        in_specs=[pl.BlockSpec((D_STYLE, TCOLS), lambda i: (0, i))],
        out_specs=pl.BlockSpec((TCOLS, 2 * D_STYLE), lambda i: (i, 0)),
        out_shape=jax.ShapeDtypeStruct((N_STYLE, 2 * D_STYLE), jnp.float32),
    )(lut_t)

    mesh = plsc.VectorSubcoreMesh(
        core_axis_name="c", subcore_axis_name="s", num_cores=NC,
        num_subcores=NS)
    gather_k = pl.kernel(
        _gather_body,
        out_type=jax.ShapeDtypeStruct((NIDX, 2 * D_STYLE), jnp.float32),
        mesh=mesh,
        scratch_types=[
            pltpu.VMEM((N_PER_W,), jnp.int32),
            pltpu.VMEM((CHUNK, 2 * D_STYLE), jnp.float32),
            pltpu.SemaphoreType.DMA,
        ],
        compiler_params=pltpu.CompilerParams(needs_layout_passes=False),
    )
    out_pad = gather_k(x_flat, table)
    return out_pad.reshape(BATCH, HIST, 2 * D_STYLE)[:, :, :D_STYLE]


def kernel(x, lut):
    x_flat = x.reshape(NIDX).astype(jnp.int32)
    return _embed(x_flat, lut.T)


# speedup vs baseline: 2.2998x; 1.0473x over previous
import jax
import jax.numpy as jnp
from jax import lax
from jax.experimental import pallas as pl
from jax.experimental.pallas import tpu as pltpu
from jax.experimental.pallas import tpu_sc as plsc

N_STYLE = 1000000
D_STYLE = 64
BATCH = 16384
HIST = 20

NC = 2
NS = 16
NW = NC * NS
LANES = 16

NIDX = BATCH * HIST
N_PER_W = NIDX // NW

CHUNK = 256
N_CHUNKS = N_PER_W // CHUNK

TCOLS = 16384
TBLKS = (N_STYLE + TCOLS - 1) // TCOLS


def _relayout_body(lut_t_ref, table_ref):
    row = lax.broadcasted_iota(jnp.int32, (D_STYLE, 2 * D_STYLE), 0)
    col = lax.broadcasted_iota(jnp.int32, (D_STYLE, 2 * D_STYLE), 1)
    eye = (row == col).astype(jnp.float32)
    x = lut_t_ref[...]
    hi = x.astype(jnp.bfloat16).astype(jnp.float32)
    lo = x - hi
    dims = (((0,), (0,)), ((), ()))
    table_ref[...] = (
        lax.dot_general(hi, eye, dims, preferred_element_type=jnp.float32)
        + lax.dot_general(lo, eye, dims, preferred_element_type=jnp.float32))


def _gather_body(idx_hbm, table_hbm, out_hbm, idx_v, rows_v, sem):
    wid = lax.axis_index("s") * NC + lax.axis_index("c")
    base = wid * N_PER_W
    pltpu.sync_copy(idx_hbm.at[pl.ds(base, N_PER_W)], idx_v)

    def chunk_step(k, carry):
        off = k * CHUNK
        idx_chunk = idx_v.at[pl.ds(off, CHUNK)]
        pltpu.async_copy(table_hbm.at[idx_chunk], rows_v, sem).wait()
        pltpu.sync_copy(rows_v, out_hbm.at[pl.ds(base + off, CHUNK)])
        return carry

    lax.fori_loop(0, N_CHUNKS, chunk_step, 0)


@jax.jit
def _embed(x_flat, lut_t):
    table = pl.pallas_call(
        _relayout_body,
        grid=(TBLKS,),
        in_specs=[pl.BlockSpec((D_STYLE, TCOLS), lambda i: (0, i))],
        out_specs=pl.BlockSpec((TCOLS, 2 * D_STYLE), lambda i: (i, 0)),
        out_shape=jax.ShapeDtypeStruct((N_STYLE, 2 * D_STYLE), jnp.float32),
    )(lut_t)

    mesh = plsc.VectorSubcoreMesh(
        core_axis_name="c", subcore_axis_name="s", num_cores=NC,
        num_subcores=NS)
    gather_k = pl.kernel(
        _gather_body,
        out_type=jax.ShapeDtypeStruct((NIDX, 2 * D_STYLE), jnp.float32),
        mesh=mesh,
        scratch_types=[
            pltpu.VMEM((N_PER_W,), jnp.int32),
            pltpu.VMEM((CHUNK, 2 * D_STYLE), jnp.float32),
            pltpu.SemaphoreType.DMA,
        ],
        compiler_params=pltpu.CompilerParams(needs_layout_passes=False),
    )
    out_pad = gather_k(x_flat, table)
    return out_pad.reshape(BATCH, HIST, 2 * D_STYLE)[:, :, :D_STYLE]


def kernel(x, lut):
    x_flat = x.reshape(NIDX).astype(jnp.int32)
    return _embed(x_flat, lut.T)
